# Initial kernel scaffold; baseline (speedup 1.0000x reference)
import jax
import jax.numpy as jnp
from jax.experimental import pallas as pl


def kernel(content_x, social_x, content_edge_index, social_edge_index, cW0, cas0, cad0, cb0, cW1, cas1, cad1, cb1, sW0, sas0, sad0, sb0, sW1, sas1, sad1, sb1, att_cw, att_cb, att_sw, att_sb, fc1_w, fc1_b, fc2_w, fc2_b):
    raise NotImplementedError("write your pallas kernel here")


# speedup vs baseline: 1.0033x; 1.0033x over previous
import functools

import jax
import jax.numpy as jnp
from jax.experimental import pallas as pl
from jax.experimental.pallas import tpu as pltpu

N = 10000
E = 320000
D = 128
H = 8
C = 16
HID = H * C
OUT = 40


def _dense_proj_kernel(x_ref, w_ref, as_ref, ad_ref, h_ref, asrc_ref, adst_ref):
    h = jnp.dot(x_ref[...], w_ref[...], preferred_element_type=jnp.float32)
    h_ref[...] = h
    asrc_ref[...] = jnp.dot(h, as_ref[...], preferred_element_type=jnp.float32)
    adst_ref[...] = jnp.dot(h, ad_ref[...], preferred_element_type=jnp.float32)


def _dense_proj(x, W, a_src, a_dst):
    eye = jnp.eye(H, dtype=jnp.float32)
    As = (a_src.reshape(H, C)[:, :, None] * eye[:, None, :]).reshape(HID, H)
    Ad = (a_dst.reshape(H, C)[:, :, None] * eye[:, None, :]).reshape(HID, H)
    din = x.shape[1]
    return pl.pallas_call(
        _dense_proj_kernel,
        out_shape=(
            jax.ShapeDtypeStruct((N, HID), jnp.float32),
            jax.ShapeDtypeStruct((N, H), jnp.float32),
            jax.ShapeDtypeStruct((N, H), jnp.float32),
        ),
    )(x, W, As, Ad)


def _edge_phase(h, asrc, adst, src, dst):
    e = asrc[src] + adst[dst]
    e = jax.nn.leaky_relu(e, 0.2)
    m = jax.ops.segment_max(e, dst, num_segments=N)
    m = jnp.where(jnp.isfinite(m), m, 0.0)
    ex = jnp.exp(e - m[dst])
    denom = jax.ops.segment_sum(ex, dst, num_segments=N)
    alpha = ex / (denom[dst] + 1e-16)
    hh = h.reshape(N, H, C)
    out = jax.ops.segment_sum(hh[src] * alpha[:, :, None], dst, num_segments=N)
    return out.reshape(N, HID)


def _gat_layer(x, src, dst, W, a_src, a_dst, b):
    h, asrc, adst = _dense_proj(x, W, a_src, a_dst)
    out = _edge_phase(h, asrc, adst, src, dst)
    return jax.nn.elu(out + b)


def _head_kernel(cx_ref, sx_ref, acw_ref, asw_ref, ab_ref,
                 fc1w_ref, fc1b_ref, fc2w_ref, fc2b_ref, o_ref):
    cx = cx_ref[...]
    sx = sx_ref[...]
    c_score = jax.nn.sigmoid(
        jnp.dot(cx, acw_ref[...], preferred_element_type=jnp.float32)
        + ab_ref[0, 0])
    s_score = jax.nn.sigmoid(
        jnp.dot(sx, asw_ref[...], preferred_element_type=jnp.float32)
        + ab_ref[0, 1])
    mx = jnp.maximum(c_score, s_score)
    ec = jnp.exp(c_score - mx)
    es = jnp.exp(s_score - mx)
    z = ec + es
    x1 = jnp.dot(cx * (ec / z), fc1w_ref[0], preferred_element_type=jnp.float32)
    x2 = jnp.dot(sx * (es / z), fc1w_ref[1], preferred_element_type=jnp.float32)
    s = x1 + x2 + fc1b_ref[...]
    x = jnp.where(s > 0, s, jnp.exp(jnp.minimum(s, 0.0)) - 1.0)
    y = jnp.dot(x, fc2w_ref[...], preferred_element_type=jnp.float32) + fc2b_ref[...]
    my = jnp.max(y, axis=1, keepdims=True)
    ey = jnp.exp(y - my)
    o_ref[...] = (y - my) - jnp.log(jnp.sum(ey, axis=1, keepdims=True))


def _head(cx, sx, att_cw, att_cb, att_sw, att_sb, fc1_w, fc1_b, fc2_w, fc2_b):
    ab = jnp.stack([att_cb, att_sb], axis=1)
    fc1 = fc1_w.reshape(2, HID, 16)
    return pl.pallas_call(
        _head_kernel,
        out_shape=jax.ShapeDtypeStruct((N, OUT), jnp.float32),
    )(cx, sx, att_cw, att_sw, ab, fc1, fc1_b, fc2_w, fc2_b)


def kernel(content_x, social_x, content_edge_index, social_edge_index,
           cW0, cas0, cad0, cb0, cW1, cas1, cad1, cb1,
           sW0, sas0, sad0, sb0, sW1, sas1, sad1, sb1,
           att_cw, att_cb, att_sw, att_sb,
           fc1_w, fc1_b, fc2_w, fc2_b):
    csrc, cdst = content_edge_index[0], content_edge_index[1]
    ssrc, sdst = social_edge_index[0], social_edge_index[1]
    cx = _gat_layer(content_x, csrc, cdst, cW0, cas0, cad0, cb0)
    cx = _gat_layer(cx, csrc, cdst, cW1, cas1, cad1, cb1)
    sx = _gat_layer(social_x, ssrc, sdst, sW0, sas0, sad0, sb0)
    sx = _gat_layer(sx, ssrc, sdst, sW1, sas1, sad1, sb1)
    return _head(cx, sx, att_cw, att_cb, att_sw, att_sb,
                 fc1_w, fc1_b, fc2_w, fc2_b)


# speedup vs baseline: 39.3642x; 39.2366x over previous
import jax
import jax.numpy as jnp
from jax import lax
from jax.experimental import pallas as pl
from jax.experimental.pallas import tpu as pltpu
from jax.experimental.pallas import tpu_sc as plsc

N = 10000
E = 320000
D = 128
H = 8
C = 16
HID = H * C
OUT = 40

NC = 2
NS = 16
CH = 512
NCHUNK = E // CH
MAXJ = -(-NCHUNK // NS)
HPC = H // NC
FPC = HID // NC

_SC_PARAMS = pltpu.CompilerParams(use_tc_tiling_on_sc=False,
                                  needs_layout_passes=False)


def _iota16():
    return lax.iota(jnp.int32, 16)


def _full16(v):
    return jnp.full((16,), v, jnp.int32)


def _lane_bcast(x, j):
    dn = lax.GatherDimensionNumbers(
        offset_dims=(), collapsed_slice_dims=(0,), start_index_map=(0,))
    return lax.gather(x, _full16(j)[:, None], dn, (1,),
                      mode=lax.GatherScatterMode.PROMISE_IN_BOUNDS)


def _elu(x):
    return jnp.where(x > 0, x, jnp.exp(jnp.minimum(x, 0.0)) - 1.0)


def _first_proj_kernel(x_ref, w_ref, as_ref, ad_ref,
                       h_ref, asrc_ref, adst_ref, m_ref):
    h2 = jnp.dot(x_ref[...], w_ref[...], preferred_element_type=jnp.float32)
    h_ref[...] = h2
    asrc = jnp.dot(h2, as_ref[...], preferred_element_type=jnp.float32)
    adst = jnp.dot(h2, ad_ref[...], preferred_element_type=jnp.float32)
    asrc_ref[...] = asrc
    adst_ref[...] = adst
    s = jnp.max(asrc, axis=0) + jnp.max(adst, axis=0)
    m8 = jnp.where(s > 0, s, 0.2 * s)
    m_ref[...] = jnp.broadcast_to(m8[:, None], (H, 128))


def _mid_proj_kernel(x_ref, b_ref, w_ref, as_ref, ad_ref,
                     h_ref, asrc_ref, adst_ref, m_ref):
    x = _elu(x_ref[...] + b_ref[...])
    h2 = jnp.dot(x, w_ref[...], preferred_element_type=jnp.float32)
    h_ref[...] = h2
    asrc = jnp.dot(h2, as_ref[...], preferred_element_type=jnp.float32)
    adst = jnp.dot(h2, ad_ref[...], preferred_element_type=jnp.float32)
    asrc_ref[...] = asrc
    adst_ref[...] = adst
    s = jnp.max(asrc, axis=0) + jnp.max(adst, axis=0)
    m8 = jnp.where(s > 0, s, 0.2 * s)
    m_ref[...] = jnp.broadcast_to(m8[:, None], (H, 128))


def _attn_mats(a_src, a_dst):
    eye = jnp.eye(H, dtype=jnp.float32)
    As = (a_src.reshape(H, C)[:, :, None] * eye[:, None, :]).reshape(HID, H)
    Ad = (a_dst.reshape(H, C)[:, :, None] * eye[:, None, :]).reshape(HID, H)
    return As, Ad


_PROJ_OUT = (
    jax.ShapeDtypeStruct((N, HID), jnp.float32),
    jax.ShapeDtypeStruct((N, H), jnp.float32),
    jax.ShapeDtypeStruct((N, H), jnp.float32),
    jax.ShapeDtypeStruct((H, 128), jnp.float32),
)


def _split_proj(h2, asrc, adst, m):
    hsplit = h2.reshape(N, NC, 2, 32).transpose(1, 2, 0, 3)
    apack = jnp.stack([
        jnp.concatenate([asrc[:, :HPC], adst[:, :HPC]], axis=1),
        jnp.concatenate([asrc[:, HPC:], adst[:, HPC:]], axis=1)])
    msp = m[:, :16].reshape(NC, HPC, 16)
    return hsplit, apack, msp


def _first_proj(x, W, a_src, a_dst):
    As, Ad = _attn_mats(a_src, a_dst)
    h2, asrc, adst, m = pl.pallas_call(
        _first_proj_kernel, out_shape=_PROJ_OUT)(x, W, As, Ad)
    return _split_proj(h2, asrc, adst, m)


def _mid_proj(acc, b, W, a_src, a_dst):
    As, Ad = _attn_mats(a_src, a_dst)
    x = acc.transpose(2, 0, 1, 3).reshape(N, HID)
    h2, asrc, adst, m = pl.pallas_call(
        _mid_proj_kernel, out_shape=_PROJ_OUT)(x, b, W, As, Ad)
    return _split_proj(h2, asrc, adst, m)


def _head_kernel(cxl_ref, cb_ref, sxl_ref, sb_ref, acw_ref, asw_ref,
                 ab_ref, fc1w_ref, fc1b_ref, fc2w_ref, fc2b_ref, o_ref):
    cx = _elu(cxl_ref[...] + cb_ref[...])
    sx = _elu(sxl_ref[...] + sb_ref[...])
    c_score = jax.nn.sigmoid(
        jnp.dot(cx, acw_ref[...], preferred_element_type=jnp.float32)
        + ab_ref[0, 0])
    s_score = jax.nn.sigmoid(
        jnp.dot(sx, asw_ref[...], preferred_element_type=jnp.float32)
        + ab_ref[0, 1])
    mx = jnp.maximum(c_score, s_score)
    ec = jnp.exp(c_score - mx)
    es = jnp.exp(s_score - mx)
    z = ec + es
    x1 = jnp.dot(cx * (ec / z), fc1w_ref[0], preferred_element_type=jnp.float32)
    x2 = jnp.dot(sx * (es / z), fc1w_ref[1], preferred_element_type=jnp.float32)
    x = _elu(x1 + x2 + fc1b_ref[...])
    y = jnp.dot(x, fc2w_ref[...], preferred_element_type=jnp.float32) + fc2b_ref[...]
    my = jnp.max(y, axis=1, keepdims=True)
    ey = jnp.exp(y - my)
    o_ref[...] = (y - my) - jnp.log(jnp.sum(ey, axis=1, keepdims=True))


def _head(cacc, cb, sacc, sb, att_cw, att_cb, att_sw, att_sb,
          fc1_w, fc1_b, fc2_w, fc2_b):
    cxl = cacc.transpose(2, 0, 1, 3).reshape(N, HID)
    sxl = sacc.transpose(2, 0, 1, 3).reshape(N, HID)
    ab = jnp.stack([att_cb, att_sb], axis=1)
    fc1 = fc1_w.reshape(2, HID, 16)
    return pl.pallas_call(
        _head_kernel,
        out_shape=jax.ShapeDtypeStruct((N, OUT), jnp.float32),
    )(cxl, cb, sxl, sb, att_cw, att_sw, ab, fc1, fc1_b, fc2_w, fc2_b)


_MESH = plsc.VectorSubcoreMesh(core_axis_name="c", subcore_axis_name="s",
                               num_cores=NC, num_subcores=NS)


def _sc_a_body(edge, apack, mhbm, ex_out, den_out, den_part,
               apack_v, m_v, src_v, dst_v, ex_v, den_t, red_a, red_b):
    c = lax.axis_index("c")
    s = lax.axis_index("s")

    pltpu.sync_copy(apack.at[c], apack_v)
    pltpu.sync_copy(mhbm.at[c], m_v)

    z16 = jnp.zeros((16,), jnp.float32)

    def zbody(i, _):
        den_t[pl.ds(i * 16, 16)] = z16
        return 0
    lax.fori_loop(0, N * HPC // 16, zbody, 0)

    mh = [m_v[h, :] for h in range(HPC)]
    iota = _iota16()

    def chunk_a(j, _):
        k = s + NS * j

        @pl.when(k < NCHUNK)
        def _():
            base = k * CH
            pltpu.sync_copy(edge.at[0, pl.ds(base, CH)], src_v)
            pltpu.sync_copy(edge.at[1, pl.ds(base, CH)], dst_v)

            def body(i, e16):
                src16 = src_v[pl.ds(i * 16, 16)]
                dst16 = dst_v[pl.ds(i * 16, 16)]
                dst4 = dst16 * 4
                for h in range(HPC):
                    es = plsc.load_gather(apack_v, [src16, _full16(h)])
                    ed = plsc.load_gather(apack_v, [dst16, _full16(HPC + h)])
                    sv = es + ed
                    sv = jnp.where(sv > 0, sv, 0.2 * sv)
                    exh = jnp.exp(sv - mh[h])
                    plsc.store_scatter(ex_v, [e16, _full16(h)], exh)
                    plsc.addupdate_scatter(den_t, [dst4 + h], exh)
                return e16 + 16
            lax.fori_loop(0, CH // 16, body, iota)

            pltpu.sync_copy(ex_v, ex_out.at[c, pl.ds(base, CH), :])
        return 0
    lax.fori_loop(0, MAXJ, chunk_a, 0)

    pltpu.sync_copy(den_t, den_part.at[c, s])
    plsc.subcore_barrier()

    r0 = s * 640 * HPC
    nv = jnp.where(s == NS - 1, 400 * HPC, 640 * HPC)
    @pl.when(s < NS - 1)
    def _():
        pltpu.sync_copy(den_part.at[c, 0, pl.ds(r0, 640 * HPC)], red_a)
        for p in range(1, NS):
            pltpu.sync_copy(den_part.at[c, p, pl.ds(r0, 640 * HPC)], red_b)

            def radd(i, _):
                red_a[pl.ds(i * 16, 16)] = (red_a[pl.ds(i * 16, 16)]
                                            + red_b[pl.ds(i * 16, 16)])
                return 0
            lax.fori_loop(0, 640 * HPC // 16, radd, 0)
        pltpu.sync_copy(red_a, den_out.at[c, pl.ds(r0, 640 * HPC)])

    @pl.when(s == NS - 1)
    def _():
        pltpu.sync_copy(den_part.at[c, 0, pl.ds(r0, 400 * HPC)],
                        red_a.at[pl.ds(0, 400 * HPC)])
        for p in range(1, NS):
            pltpu.sync_copy(den_part.at[c, p, pl.ds(r0, 400 * HPC)],
                            red_b.at[pl.ds(0, 400 * HPC)])

            def radd(i, _):
                red_a[pl.ds(i * 16, 16)] = (red_a[pl.ds(i * 16, 16)]
                                            + red_b[pl.ds(i * 16, 16)])
                return 0
            lax.fori_loop(0, 400 * HPC // 16, radd, 0)
        pltpu.sync_copy(red_a.at[pl.ds(0, 400 * HPC)],
                        den_out.at[c, pl.ds(r0, 400 * HPC)])


_sc_a_call = pl.kernel(
    _sc_a_body,
    out_type=(
        jax.ShapeDtypeStruct((NC, E, HPC), jnp.float32),
        jax.ShapeDtypeStruct((NC, N * HPC), jnp.float32),
        jax.ShapeDtypeStruct((NC, NS, N * HPC), jnp.float32),
    ),
    mesh=_MESH,
    compiler_params=_SC_PARAMS,
    scratch_types=[
        pltpu.VMEM((N, 2 * HPC), jnp.float32),
        pltpu.VMEM((HPC, 16), jnp.float32),
        pltpu.VMEM((CH,), jnp.int32),
        pltpu.VMEM((CH,), jnp.int32),
        pltpu.VMEM((CH, HPC), jnp.float32),
        pltpu.VMEM((N * HPC,), jnp.float32),
        pltpu.VMEM((640 * HPC,), jnp.float32),
        pltpu.VMEM((640 * HPC,), jnp.float32),
    ],
)


def _sc_b_body(edge, exhbm, denhbm, hsplit, acc_out,
               den_v, src_v, dst_v, dst2_v, ex_v, hrows_v, acc_sp, sem):
    c = lax.axis_index("c")
    s = lax.axis_index("s")

    pltpu.sync_copy(denhbm.at[c], den_v)

    z16 = jnp.zeros((16,), jnp.float32)
    iota = _iota16()

    for half in range(2):
        def zbody(r, _):
            for cc in range(2):
                hrows_v[r, pl.ds(cc * 16, 16)] = z16
            return 0
        lax.fori_loop(0, CH, zbody, 0)

        @pl.when(s < 9)
        def _():
            pltpu.sync_copy(hrows_v, acc_sp.at[pl.ds(s * 1024, CH), :])
            pltpu.sync_copy(hrows_v, acc_sp.at[pl.ds(s * 1024 + CH, CH), :])

        @pl.when(s == 9)
        def _():
            nr = N - 9 * 1024 - CH
            pltpu.sync_copy(hrows_v, acc_sp.at[pl.ds(9 * 1024, CH), :])
            pltpu.sync_copy(hrows_v.at[pl.ds(0, nr), :],
                            acc_sp.at[pl.ds(9 * 1024 + CH, nr), :])

        plsc.subcore_barrier()

        hv = hsplit.at[c, half]

        def chunk_b(j, _):
            k = s + NS * j

            @pl.when(k < NCHUNK)
            def _():
                base = k * CH
                pltpu.sync_copy(edge.at[0, pl.ds(base, CH)], src_v)
                pltpu.sync_copy(edge.at[1, pl.ds(base, CH)], dst_v)
                pltpu.sync_copy(exhbm.at[c, pl.ds(base, CH), :], ex_v)
                for jj in range(4):
                    pltpu.async_copy(
                        hv.at[src_v.at[pl.ds(jj * 128, 128)]],
                        hrows_v.at[pl.ds(jj * 128, 128), :], sem).wait()

                def body(i, e16):
                    dst16 = dst_v[pl.ds(i * 16, 16)]
                    dst4 = dst16 * 4
                    al = []
                    for hh in range(2):
                        ha = half * 2 + hh
                        exh = plsc.load_gather(ex_v, [e16, _full16(ha)])
                        dh = plsc.load_gather(den_v, [dst4 + ha])
                        al.append(exh / (dh + 1e-16))
                    for jl in range(16):
                        row = i * 16 + jl
                        for hh in range(2):
                            sp = _lane_bcast(al[hh], jl)
                            hrows_v[row, pl.ds(hh * 16, 16)] = (
                                hrows_v[row, pl.ds(hh * 16, 16)] * sp)
                    return e16 + 16
                lax.fori_loop(0, CH // 16, body, iota)

                def scat(i, _):
                    dst16 = dst_v[pl.ds(i * 16, 16)]
                    pltpu.sync_copy(hrows_v.at[pl.ds(i * 16, 16), :],
                                    acc_sp.at[dst16], add=True)
                    return 0
                lax.fori_loop(0, CH // 16, scat, 0)
            return 0
        lax.fori_loop(0, MAXJ, chunk_b, 0)

        plsc.subcore_barrier()

        @pl.when(s < 9)
        def _():
            pltpu.sync_copy(acc_sp.at[pl.ds(s * 1024, 1024), :],
                            acc_out.at[c, half, pl.ds(s * 1024, 1024), :])

        @pl.when(s == 9)
        def _():
            nr = N - 9 * 1024
            pltpu.sync_copy(acc_sp.at[pl.ds(9 * 1024, nr), :],
                            acc_out.at[c, half, pl.ds(9 * 1024, nr), :])

        plsc.subcore_barrier()


_sc_b_call = pl.kernel(
    _sc_b_body,
    out_type=jax.ShapeDtypeStruct((NC, 2, N, 32), jnp.float32),
    mesh=_MESH,
    compiler_params=_SC_PARAMS,
    scratch_types=[
        pltpu.VMEM((N * HPC,), jnp.float32),
        pltpu.VMEM((CH,), jnp.int32),
        pltpu.VMEM((CH,), jnp.int32),
        pltpu.VMEM((4, 128), jnp.int32),
        pltpu.VMEM((CH, HPC), jnp.float32),
        pltpu.VMEM((CH, 32), jnp.float32),
        pltpu.VMEM_SHARED((N, 32), jnp.float32),
        pltpu.SemaphoreType.DMA,
    ],
)


def _gat_edge(edge, hsplit, apack, m):
    ex, den, _ = _sc_a_call(edge, apack, m)
    return _sc_b_call(edge, ex, den, hsplit)


def kernel(content_x, social_x, content_edge_index, social_edge_index,
           cW0, cas0, cad0, cb0, cW1, cas1, cad1, cb1,
           sW0, sas0, sad0, sb0, sW1, sas1, sad1, sb1,
           att_cw, att_cb, att_sw, att_sb,
           fc1_w, fc1_b, fc2_w, fc2_b):
    ce = content_edge_index
    se = social_edge_index

    chs, cap, cm = _first_proj(content_x, cW0, cas0, cad0)
    cacc = _gat_edge(ce, chs, cap, cm)
    chs, cap, cm = _mid_proj(cacc, cb0, cW1, cas1, cad1)
    cacc = _gat_edge(ce, chs, cap, cm)

    shs, sap, sm = _first_proj(social_x, sW0, sas0, sad0)
    sacc = _gat_edge(se, shs, sap, sm)
    shs, sap, sm = _mid_proj(sacc, sb0, sW1, sas1, sad1)
    sacc = _gat_edge(se, shs, sap, sm)

    return _head(cacc, cb1, sacc, sb1, att_cw, att_cb, att_sw, att_sb,
                 fc1_w, fc1_b, fc2_w, fc2_b)


# speedup vs baseline: 40.8049x; 1.0366x over previous
import jax
import jax.numpy as jnp
from jax import lax
from jax.experimental import pallas as pl
from jax.experimental.pallas import tpu as pltpu
from jax.experimental.pallas import tpu_sc as plsc

N = 10000
E = 320000
D = 128
H = 8
C = 16
HID = H * C
OUT = 40

NC = 2
NS = 16
CH = 512
NCHUNK = E // CH
MAXJ = -(-NCHUNK // NS)
HPC = H // NC
FPC = HID // NC

_SC_PARAMS = pltpu.CompilerParams(use_tc_tiling_on_sc=False,
                                  needs_layout_passes=False)


def _iota16():
    return lax.iota(jnp.int32, 16)


def _full16(v):
    return jnp.full((16,), v, jnp.int32)


def _lane_bcast(x, j):
    dn = lax.GatherDimensionNumbers(
        offset_dims=(), collapsed_slice_dims=(0,), start_index_map=(0,))
    return lax.gather(x, _full16(j)[:, None], dn, (1,),
                      mode=lax.GatherScatterMode.PROMISE_IN_BOUNDS)


def _elu(x):
    return jnp.where(x > 0, x, jnp.exp(jnp.minimum(x, 0.0)) - 1.0)


def _first_proj_kernel(x_ref, w_ref, as_ref, ad_ref,
                       h_ref, asrc_ref, adst_ref, m_ref):
    h2 = jnp.dot(x_ref[...], w_ref[...], preferred_element_type=jnp.float32)
    h_ref[...] = h2
    asrc = jnp.dot(h2, as_ref[...], preferred_element_type=jnp.float32)
    adst = jnp.dot(h2, ad_ref[...], preferred_element_type=jnp.float32)
    asrc_ref[...] = asrc
    adst_ref[...] = adst
    s = jnp.max(asrc, axis=0) + jnp.max(adst, axis=0)
    m8 = jnp.where(s > 0, s, 0.2 * s)
    m_ref[...] = jnp.broadcast_to(m8[:, None], (H, 128))


def _mid_proj_kernel(x_ref, b_ref, w_ref, as_ref, ad_ref,
                     h_ref, asrc_ref, adst_ref, m_ref):
    x = _elu(x_ref[...] + b_ref[...])
    h2 = jnp.dot(x, w_ref[...], preferred_element_type=jnp.float32)
    h_ref[...] = h2
    asrc = jnp.dot(h2, as_ref[...], preferred_element_type=jnp.float32)
    adst = jnp.dot(h2, ad_ref[...], preferred_element_type=jnp.float32)
    asrc_ref[...] = asrc
    adst_ref[...] = adst
    s = jnp.max(asrc, axis=0) + jnp.max(adst, axis=0)
    m8 = jnp.where(s > 0, s, 0.2 * s)
    m_ref[...] = jnp.broadcast_to(m8[:, None], (H, 128))


def _attn_mats(a_src, a_dst):
    eye = jnp.eye(H, dtype=jnp.float32)
    As = (a_src.reshape(H, C)[:, :, None] * eye[:, None, :]).reshape(HID, H)
    Ad = (a_dst.reshape(H, C)[:, :, None] * eye[:, None, :]).reshape(HID, H)
    return As, Ad


_PROJ_OUT = (
    jax.ShapeDtypeStruct((N, HID), jnp.float32),
    jax.ShapeDtypeStruct((N, H), jnp.float32),
    jax.ShapeDtypeStruct((N, H), jnp.float32),
    jax.ShapeDtypeStruct((H, 128), jnp.float32),
)


def _split_proj(h2, asrc, adst, m):
    hsplit = h2.reshape(N, NC, 2, 32).transpose(1, 2, 0, 3)
    apack = jnp.stack([
        jnp.concatenate([asrc[:, :HPC], adst[:, :HPC]], axis=1),
        jnp.concatenate([asrc[:, HPC:], adst[:, HPC:]], axis=1)])
    msp = m[:, :16].reshape(NC, HPC, 16)
    return hsplit, apack, msp


def _first_proj(x, W, a_src, a_dst):
    As, Ad = _attn_mats(a_src, a_dst)
    h2, asrc, adst, m = pl.pallas_call(
        _first_proj_kernel, out_shape=_PROJ_OUT)(x, W, As, Ad)
    return _split_proj(h2, asrc, adst, m)


def _mid_proj(acc, b, W, a_src, a_dst):
    As, Ad = _attn_mats(a_src, a_dst)
    x = acc.transpose(2, 0, 1, 3).reshape(N, HID)
    h2, asrc, adst, m = pl.pallas_call(
        _mid_proj_kernel, out_shape=_PROJ_OUT)(x, b, W, As, Ad)
    return _split_proj(h2, asrc, adst, m)


def _head_kernel(cxl_ref, cb_ref, sxl_ref, sb_ref, acw_ref, asw_ref,
                 ab_ref, fc1w_ref, fc1b_ref, fc2w_ref, fc2b_ref, o_ref):
    cx = _elu(cxl_ref[...] + cb_ref[...])
    sx = _elu(sxl_ref[...] + sb_ref[...])
    c_score = jax.nn.sigmoid(
        jnp.dot(cx, acw_ref[...], preferred_element_type=jnp.float32)
        + ab_ref[0, 0])
    s_score = jax.nn.sigmoid(
        jnp.dot(sx, asw_ref[...], preferred_element_type=jnp.float32)
        + ab_ref[0, 1])
    mx = jnp.maximum(c_score, s_score)
    ec = jnp.exp(c_score - mx)
    es = jnp.exp(s_score - mx)
    z = ec + es
    x1 = jnp.dot(cx * (ec / z), fc1w_ref[0], preferred_element_type=jnp.float32)
    x2 = jnp.dot(sx * (es / z), fc1w_ref[1], preferred_element_type=jnp.float32)
    x = _elu(x1 + x2 + fc1b_ref[...])
    y = jnp.dot(x, fc2w_ref[...], preferred_element_type=jnp.float32) + fc2b_ref[...]
    my = jnp.max(y, axis=1, keepdims=True)
    ey = jnp.exp(y - my)
    o_ref[...] = (y - my) - jnp.log(jnp.sum(ey, axis=1, keepdims=True))


def _head(cacc, cb, sacc, sb, att_cw, att_cb, att_sw, att_sb,
          fc1_w, fc1_b, fc2_w, fc2_b):
    cxl = cacc.transpose(2, 0, 1, 3).reshape(N, HID)
    sxl = sacc.transpose(2, 0, 1, 3).reshape(N, HID)
    ab = jnp.stack([att_cb, att_sb], axis=1)
    fc1 = fc1_w.reshape(2, HID, 16)
    return pl.pallas_call(
        _head_kernel,
        out_shape=jax.ShapeDtypeStruct((N, OUT), jnp.float32),
    )(cxl, cb, sxl, sb, att_cw, att_sw, ab, fc1, fc1_b, fc2_w, fc2_b)


_MESH = plsc.VectorSubcoreMesh(core_axis_name="c", subcore_axis_name="s",
                               num_cores=NC, num_subcores=NS)


def _sc_a_body(edge, apack, mhbm, ex_out, den_out, den_part,
               apack_v, m_v, src_v, dst_v, ex_v, den_t, red_a, red_b):
    c = lax.axis_index("c")
    s = lax.axis_index("s")

    pltpu.sync_copy(apack.at[c], apack_v)
    pltpu.sync_copy(mhbm.at[c], m_v)

    z16 = jnp.zeros((16,), jnp.float32)

    def zbody(i, _):
        den_t[pl.ds(i * 16, 16)] = z16
        return 0
    lax.fori_loop(0, N * HPC // 16, zbody, 0)

    mh = [m_v[h, :] for h in range(HPC)]
    iota = _iota16()

    def chunk_a(j, _):
        k = s + NS * j

        @pl.when(k < NCHUNK)
        def _():
            base = k * CH
            pltpu.sync_copy(edge.at[0, pl.ds(base, CH)], src_v)
            pltpu.sync_copy(edge.at[1, pl.ds(base, CH)], dst_v)

            def body(i, e16):
                src16 = src_v[pl.ds(i * 16, 16)]
                dst16 = dst_v[pl.ds(i * 16, 16)]
                dst4 = dst16 * 4
                for h in range(HPC):
                    es = plsc.load_gather(apack_v, [src16, _full16(h)])
                    ed = plsc.load_gather(apack_v, [dst16, _full16(HPC + h)])
                    sv = es + ed
                    sv = jnp.where(sv > 0, sv, 0.2 * sv)
                    exh = jnp.exp(sv - mh[h])
                    plsc.store_scatter(ex_v, [e16, _full16(h)], exh)
                    plsc.addupdate_scatter(den_t, [dst4 + h], exh)
                return e16 + 16
            lax.fori_loop(0, CH // 16, body, iota)

            pltpu.sync_copy(ex_v, ex_out.at[c, pl.ds(base, CH), :])
        return 0
    lax.fori_loop(0, MAXJ, chunk_a, 0)

    pltpu.sync_copy(den_t, den_part.at[c, s])
    plsc.subcore_barrier()

    r0 = s * 640 * HPC
    nv = jnp.where(s == NS - 1, 400 * HPC, 640 * HPC)
    @pl.when(s < NS - 1)
    def _():
        pltpu.sync_copy(den_part.at[c, 0, pl.ds(r0, 640 * HPC)], red_a)
        for p in range(1, NS):
            pltpu.sync_copy(den_part.at[c, p, pl.ds(r0, 640 * HPC)], red_b)

            def radd(i, _):
                red_a[pl.ds(i * 16, 16)] = (red_a[pl.ds(i * 16, 16)]
                                            + red_b[pl.ds(i * 16, 16)])
                return 0
            lax.fori_loop(0, 640 * HPC // 16, radd, 0)
        pltpu.sync_copy(red_a, den_out.at[c, pl.ds(r0, 640 * HPC)])

    @pl.when(s == NS - 1)
    def _():
        pltpu.sync_copy(den_part.at[c, 0, pl.ds(r0, 400 * HPC)],
                        red_a.at[pl.ds(0, 400 * HPC)])
        for p in range(1, NS):
            pltpu.sync_copy(den_part.at[c, p, pl.ds(r0, 400 * HPC)],
                            red_b.at[pl.ds(0, 400 * HPC)])

            def radd(i, _):
                red_a[pl.ds(i * 16, 16)] = (red_a[pl.ds(i * 16, 16)]
                                            + red_b[pl.ds(i * 16, 16)])
                return 0
            lax.fori_loop(0, 400 * HPC // 16, radd, 0)
        pltpu.sync_copy(red_a.at[pl.ds(0, 400 * HPC)],
                        den_out.at[c, pl.ds(r0, 400 * HPC)])


_sc_a_call = pl.kernel(
    _sc_a_body,
    out_type=(
        jax.ShapeDtypeStruct((NC, E, HPC), jnp.float32),
        jax.ShapeDtypeStruct((NC, N * HPC), jnp.float32),
        jax.ShapeDtypeStruct((NC, NS, N * HPC), jnp.float32),
    ),
    mesh=_MESH,
    compiler_params=_SC_PARAMS,
    scratch_types=[
        pltpu.VMEM((N, 2 * HPC), jnp.float32),
        pltpu.VMEM((HPC, 16), jnp.float32),
        pltpu.VMEM((CH,), jnp.int32),
        pltpu.VMEM((CH,), jnp.int32),
        pltpu.VMEM((CH, HPC), jnp.float32),
        pltpu.VMEM((N * HPC,), jnp.float32),
        pltpu.VMEM((640 * HPC,), jnp.float32),
        pltpu.VMEM((640 * HPC,), jnp.float32),
    ],
)


def _sc_b_body(edge, exhbm, denhbm, hsplit, acc_out,
               den_v, src_v, dst_v, dst2_v, ex_v, hrows_v, acc_sp, sem):
    c = lax.axis_index("c")
    s = lax.axis_index("s")

    pltpu.sync_copy(denhbm.at[c], den_v)

    z16 = jnp.zeros((16,), jnp.float32)
    iota = _iota16()

    for half in range(2):
        def zbody(r, _):
            for cc in range(2):
                hrows_v[r, pl.ds(cc * 16, 16)] = z16
            return 0
        lax.fori_loop(0, CH, zbody, 0)

        @pl.when(s < 9)
        def _():
            pltpu.sync_copy(hrows_v, acc_sp.at[pl.ds(s * 1024, CH), :])
            pltpu.sync_copy(hrows_v, acc_sp.at[pl.ds(s * 1024 + CH, CH), :])

        @pl.when(s == 9)
        def _():
            nr = N - 9 * 1024 - CH
            pltpu.sync_copy(hrows_v, acc_sp.at[pl.ds(9 * 1024, CH), :])
            pltpu.sync_copy(hrows_v.at[pl.ds(0, nr), :],
                            acc_sp.at[pl.ds(9 * 1024 + CH, nr), :])

        plsc.subcore_barrier()

        hv = hsplit.at[c, half]

        def chunk_b(j, _):
            k = s + NS * j

            @pl.when(k < NCHUNK)
            def _():
                base = k * CH
                pltpu.sync_copy(edge.at[0, pl.ds(base, CH)], src_v)
                pltpu.sync_copy(edge.at[1, pl.ds(base, CH)], dst_v)
                for jj in range(4):
                    pltpu.sync_copy(edge.at[1, pl.ds(base + jj * 128, 128)],
                                    dst2_v.at[jj])
                pltpu.sync_copy(exhbm.at[c, pl.ds(base, CH), :], ex_v)
                for jj in range(4):
                    pltpu.async_copy(
                        hv.at[src_v.at[pl.ds(jj * 128, 128)]],
                        hrows_v.at[pl.ds(jj * 128, 128), :], sem).wait()

                def body(i, e16):
                    dst16 = dst_v[pl.ds(i * 16, 16)]
                    dst4 = dst16 * 4
                    al = []
                    for hh in range(2):
                        ha = half * 2 + hh
                        exh = plsc.load_gather(ex_v, [e16, _full16(ha)])
                        dh = plsc.load_gather(den_v, [dst4 + ha])
                        al.append(exh / (dh + 1e-16))
                    for jl in range(16):
                        row = i * 16 + jl
                        for hh in range(2):
                            sp = _lane_bcast(al[hh], jl)
                            hrows_v[row, pl.ds(hh * 16, 16)] = (
                                hrows_v[row, pl.ds(hh * 16, 16)] * sp)
                    return e16 + 16
                lax.fori_loop(0, CH // 16, body, iota)

                for jj in range(4):
                    pltpu.sync_copy(hrows_v.at[pl.ds(jj * 128, 128), :],
                                    acc_sp.at[dst2_v.at[jj]], add=True)
            return 0
        lax.fori_loop(0, MAXJ, chunk_b, 0)

        plsc.subcore_barrier()

        @pl.when(s < 9)
        def _():
            pltpu.sync_copy(acc_sp.at[pl.ds(s * 1024, 1024), :],
                            acc_out.at[c, half, pl.ds(s * 1024, 1024), :])

        @pl.when(s == 9)
        def _():
            nr = N - 9 * 1024
            pltpu.sync_copy(acc_sp.at[pl.ds(9 * 1024, nr), :],
                            acc_out.at[c, half, pl.ds(9 * 1024, nr), :])

        plsc.subcore_barrier()


_sc_b_call = pl.kernel(
    _sc_b_body,
    out_type=jax.ShapeDtypeStruct((NC, 2, N, 32), jnp.float32),
    mesh=_MESH,
    compiler_params=_SC_PARAMS,
    scratch_types=[
        pltpu.VMEM((N * HPC,), jnp.float32),
        pltpu.VMEM((CH,), jnp.int32),
        pltpu.VMEM((CH,), jnp.int32),
        pltpu.VMEM((4, 128), jnp.int32),
        pltpu.VMEM((CH, HPC), jnp.float32),
        pltpu.VMEM((CH, 32), jnp.float32),
        pltpu.VMEM_SHARED((N, 32), jnp.float32),
        pltpu.SemaphoreType.DMA,
    ],
)


def _gat_edge(edge, hsplit, apack, m):
    ex, den, _ = _sc_a_call(edge, apack, m)
    return _sc_b_call(edge, ex, den, hsplit)


def kernel(content_x, social_x, content_edge_index, social_edge_index,
           cW0, cas0, cad0, cb0, cW1, cas1, cad1, cb1,
           sW0, sas0, sad0, sb0, sW1, sas1, sad1, sb1,
           att_cw, att_cb, att_sw, att_sb,
           fc1_w, fc1_b, fc2_w, fc2_b):
    ce = content_edge_index
    se = social_edge_index

    chs, cap, cm = _first_proj(content_x, cW0, cas0, cad0)
    cacc = _gat_edge(ce, chs, cap, cm)
    chs, cap, cm = _mid_proj(cacc, cb0, cW1, cas1, cad1)
    cacc = _gat_edge(ce, chs, cap, cm)

    shs, sap, sm = _first_proj(social_x, sW0, sas0, sad0)
    sacc = _gat_edge(se, shs, sap, sm)
    shs, sap, sm = _mid_proj(sacc, sb0, sW1, sas1, sad1)
    sacc = _gat_edge(se, shs, sap, sm)

    return _head(cacc, cb1, sacc, sb1, att_cw, att_cb, att_sw, att_sb,
                 fc1_w, fc1_b, fc2_w, fc2_b)
